# Initial kernel scaffold; baseline (speedup 1.0000x reference)
import jax
import jax.numpy as jnp
from jax.experimental import pallas as pl


def kernel(structural_features, multimodal_features, edge_index, W_in, b_in, W_self0, W_neigh0, b_sage0, gamma0, beta0, W_self1, W_neigh1, b_sage1, gamma1, beta1, W_rel, b_rel, W_c1, b_c1, W_c2, b_c2):
    raise NotImplementedError("write your pallas kernel here")


# speedup vs baseline: 4.2209x; 4.2209x over previous
import functools

import jax
import jax.numpy as jnp
from jax import lax
from jax.experimental import pallas as pl
from jax.experimental.pallas import tpu as pltpu
from jax.experimental.pallas import tpu_sc as plsc

_N = 50000
_H = 64
_NC = 2
_NS = 16
_CH = 128
_GW = 16
_ACC = 51200
_STRIPE = _ACC // _NS

_BN = 2000
_GRID = _N // _BN
_EPS = 1e-5

_SC_PARAMS = pltpu.CompilerParams(use_tc_tiling_on_sc=False)


def _make_segsum(cpt, gpt):
  mesh = plsc.VectorSubcoreMesh(core_axis_name="c", subcore_axis_name="s")
  ngrp = cpt // gpt
  scratch = [
      pltpu.VMEM((gpt, _CH), jnp.int32),
      pltpu.VMEM((gpt, _CH), jnp.int32),
      pltpu.VMEM((_CH, _GW), jnp.float32),
      pltpu.VMEM((_CH, _GW), jnp.float32),
      pltpu.VMEM_SHARED((_ACC, _GW), jnp.float32),
      pltpu.SemaphoreType.DMA,
  ]

  def body(table, src2, dst2, agg_out, src_v, dst_v, rows, zb, acc, sem):
    c = lax.axis_index("c")
    s = lax.axis_index("s")
    base = s * _STRIPE

    @pl.loop(0, _CH)
    def _(r):
      zb[r, pl.ds(0, 16)] = jnp.zeros((16,), jnp.float32)

    @pl.loop(0, _STRIPE // _CH)
    def _(r):
      pltpu.sync_copy(zb, acc.at[pl.ds(base + r * _CH, _CH)])
    plsc.subcore_barrier()

    @pl.loop(0, ngrp)
    def _(g):
      row0 = s * cpt + g * gpt
      pltpu.sync_copy(src2.at[c].at[pl.ds(row0, gpt)], src_v)
      pltpu.sync_copy(dst2.at[pl.ds(row0, gpt)], dst_v)

      @pl.loop(0, gpt)
      def _(j):
        pltpu.async_copy(table.at[src_v.at[j]], rows, sem).wait()
        pltpu.sync_copy(rows, acc.at[dst_v.at[j]], add=True)

    plsc.subcore_barrier()
    pltpu.sync_copy(acc.at[pl.ds(base, _STRIPE)],
                    agg_out.at[c].at[pl.ds(base, _STRIPE)])

  return pl.kernel(
      body, out_type=jax.ShapeDtypeStruct((_NC, _ACC, _GW), jnp.float32),
      mesh=mesh, scratch_types=scratch, compiler_params=_SC_PARAMS)


def _make_deg(cpt, gpt):
  mesh = plsc.VectorSubcoreMesh(core_axis_name="c", subcore_axis_name="s")
  ngrp = cpt // gpt
  scratch = [
      pltpu.VMEM((gpt, _CH), jnp.int32),
      pltpu.VMEM((_CH,), jnp.float32),
      pltpu.VMEM((_CH,), jnp.float32),
      pltpu.VMEM_SHARED((_ACC,), jnp.float32),
  ]

  def body(dst2, deg_out, dst_v, ones, zrow, dacc):
    c = lax.axis_index("c")
    s = lax.axis_index("s")
    base = s * _STRIPE

    @pl.loop(0, _CH // 16)
    def _(r):
      ones[pl.ds(r * 16, 16)] = jnp.ones((16,), jnp.float32)
      zrow[pl.ds(r * 16, 16)] = jnp.zeros((16,), jnp.float32)

    @pl.loop(0, _STRIPE // _CH)
    def _(r):
      pltpu.sync_copy(zrow, dacc.at[pl.ds(base + r * _CH, _CH)])
    plsc.subcore_barrier()

    @pl.loop(0, ngrp)
    def _(g):
      pltpu.sync_copy(dst2.at[pl.ds(s * cpt + g * gpt, gpt)], dst_v)

      @pl.loop(0, gpt)
      def _(j):
        @pl.when((j % 2) == c)
        def _():
          pltpu.sync_copy(ones, dacc.at[dst_v.at[j]], add=True)

    plsc.subcore_barrier()
    pltpu.sync_copy(dacc.at[pl.ds(base, _STRIPE)],
                    deg_out.at[c].at[pl.ds(base, _STRIPE)])

  return pl.kernel(
      body, out_type=jax.ShapeDtypeStruct((_NC, _ACC), jnp.float32),
      mesh=mesh, scratch_types=scratch, compiler_params=_SC_PARAMS)


def _tc1_body(s_ref, m_ref, w_ref, b_ref, h0_ref, t0_ref):
  w = w_ref[...]
  h = jnp.dot(s_ref[...], w[:_H], preferred_element_type=jnp.float32)
  h += jnp.dot(m_ref[...], w[_H:], preferred_element_type=jnp.float32)
  h = jnp.maximum(h + b_ref[...], 0.0)
  h0_ref[...] = h
  t0_ref[...] = jnp.stack([h[:, 0:16], h[:, 16:32], h[:, 32:48], h[:, 48:64]],
                          axis=0)


def _sage_dense_body(hself_ref, aa_ref, ab_ref, d_ref, ws_ref, wn_ref, b_ref,
                     s_out_ref, st_ref, *, self_is_pre):
  i = pl.program_id(0)
  aa = aa_ref[...]
  ab = ab_ref[...]
  d = d_ref[...]
  deg = jnp.maximum(d[0] + d[1], 1.0)
  hn = jnp.concatenate([aa[0], aa[1], ab[0], ab[1]], axis=1) / deg
  if self_is_pre:
    s = hself_ref[...] + hn + b_ref[...]
  else:
    s = jnp.dot(hself_ref[...], ws_ref[...],
                preferred_element_type=jnp.float32)
    s += jnp.dot(hn, wn_ref[...], preferred_element_type=jnp.float32)
    s += b_ref[...]
  s_out_ref[...] = s

  @pl.when(i == 0)
  def _():
    st_ref[...] = jnp.zeros_like(st_ref)
  st_ref[...] += jnp.stack([jnp.sum(s, axis=0), jnp.sum(s * s, axis=0)])


def _tc2b_body(s_ref, st_ref, g_ref, be_ref, ws_ref, wn_ref,
               self1_ref, t1_ref):
  st = st_ref[...]
  mean = st[0:1] / _N
  var = st[1:2] / _N - mean * mean
  inv = lax.rsqrt(var + _EPS)
  h1 = jnp.maximum((s_ref[...] - mean) * inv * g_ref[...] + be_ref[...], 0.0)
  self1_ref[...] = jnp.dot(h1, ws_ref[...],
                           preferred_element_type=jnp.float32)
  p1 = jnp.dot(h1, wn_ref[...], preferred_element_type=jnp.float32)
  t1_ref[...] = jnp.stack(
      [p1[:, 0:16], p1[:, 16:32], p1[:, 32:48], p1[:, 48:64]], axis=0)


def _tc3b_body(s_ref, st_ref, g_ref, be_ref, h0_ref, wrel_ref, brel_ref,
               wc1_ref, bc1_ref, wc2_ref, bc2_ref, out_ref):
  st = st_ref[...]
  mean = st[0:1] / _N
  var = st[1:2] / _N - mean * mean
  inv = lax.rsqrt(var + _EPS)
  h2 = jnp.maximum((s_ref[...] - mean) * inv * g_ref[...] + be_ref[...], 0.0)
  wrel = wrel_ref[...]
  hf = jnp.dot(h0_ref[...], wrel[:_H], preferred_element_type=jnp.float32)
  hf += jnp.dot(h2, wrel[_H:], preferred_element_type=jnp.float32)
  hf = jnp.maximum(hf + brel_ref[...], 0.0)
  hid = jnp.maximum(
      jnp.dot(hf, wc1_ref[...], preferred_element_type=jnp.float32)
      + bc1_ref[...], 0.0)
  out_ref[...] = (jnp.dot(hid, wc2_ref[...],
                          preferred_element_type=jnp.float32) + bc2_ref[...])


def _row_spec(width):
  return pl.BlockSpec((_BN, width), lambda i: (i, 0))


def _full_spec(shape):
  nd = len(shape)
  return pl.BlockSpec(shape, lambda i, _n=nd: (0,) * _n)


def _agg_spec():
  return pl.BlockSpec((_NC, _BN, _GW), lambda i: (0, i, 0))


def _deg_spec():
  return pl.BlockSpec((_NC, _BN, 1), lambda i: (0, i, 0))


def _table_spec():
  return pl.BlockSpec((4, _BN, _GW), lambda i: (0, i, 0))


def kernel(structural_features, multimodal_features, edge_index, W_in, b_in,
           W_self0, W_neigh0, b_sage0, gamma0, beta0, W_self1, W_neigh1,
           b_sage1, gamma1, beta1, W_rel, b_rel, W_c1, b_c1, W_c2, b_c2):
  f32 = jnp.float32
  src = edge_index[0]
  dst = edge_index[1]
  e = src.shape[0]

  cpt = -(-e // (_CH * _NS))
  cpt = -(-cpt // 4) * 4
  gpt = cpt // 4
  e_pad = cpt * _CH * _NS
  padn = e_pad - e
  srcp = jnp.concatenate([src, jnp.zeros((padn,), jnp.int32)])
  dstp = jnp.concatenate([dst, jnp.full((padn,), _N, jnp.int32)])
  src2_a = (srcp[None, :] + jnp.array([[0], [_N]], jnp.int32)).reshape(
      _NC, e_pad // _CH, _CH)
  src2_b = (srcp[None, :] + jnp.array([[2 * _N], [3 * _N]],
                                      jnp.int32)).reshape(
      _NC, e_pad // _CH, _CH)
  dst2 = dstp.reshape(e_pad // _CH, _CH)

  segsum = _make_segsum(cpt, gpt)
  degk = _make_deg(cpt, gpt)

  b_in2 = b_in[None, :]
  b_sage0_2 = b_sage0[None, :]
  b_sage1_2 = b_sage1[None, :]
  gamma0_2, beta0_2 = gamma0[None, :], beta0[None, :]
  gamma1_2, beta1_2 = gamma1[None, :], beta1[None, :]

  degp = degk(dst2)
  deg3 = degp[:, :, None]

  h0, t0 = pl.pallas_call(
      _tc1_body,
      grid=(_GRID,),
      in_specs=[_row_spec(_H), _row_spec(_H), _full_spec((2 * _H, _H)),
                _full_spec((1, _H))],
      out_specs=[_row_spec(_H), _table_spec()],
      out_shape=[jax.ShapeDtypeStruct((_N, _H), f32),
                 jax.ShapeDtypeStruct((4, _N, _GW), f32)],
  )(structural_features, multimodal_features, W_in, b_in2)
  t0f = t0.reshape(4 * _N, _GW)

  agg0a = segsum(t0f, src2_a, dst2)
  agg0b = segsum(t0f, src2_b, dst2)

  s0, st0 = pl.pallas_call(
      functools.partial(_sage_dense_body, self_is_pre=False),
      grid=(_GRID,),
      in_specs=[_row_spec(_H), _agg_spec(), _agg_spec(), _deg_spec(),
                _full_spec((_H, 2 * _H)), _full_spec((_H, 2 * _H)),
                _full_spec((1, 2 * _H))],
      out_specs=[_row_spec(2 * _H),
                 pl.BlockSpec((2, 2 * _H), lambda i: (0, 0))],
      out_shape=[jax.ShapeDtypeStruct((_N, 2 * _H), f32),
                 jax.ShapeDtypeStruct((2, 2 * _H), f32)],
  )(h0, agg0a, agg0b, deg3, W_self0, W_neigh0, b_sage0_2)

  self1, t1 = pl.pallas_call(
      _tc2b_body,
      grid=(_GRID,),
      in_specs=[_row_spec(2 * _H), _full_spec((2, 2 * _H)),
                _full_spec((1, 2 * _H)), _full_spec((1, 2 * _H)),
                _full_spec((2 * _H, _H)), _full_spec((2 * _H, _H))],
      out_specs=[_row_spec(_H), _table_spec()],
      out_shape=[jax.ShapeDtypeStruct((_N, _H), f32),
                 jax.ShapeDtypeStruct((4, _N, _GW), f32)],
  )(s0, st0, gamma0_2, beta0_2, W_self1, W_neigh1)
  t1f = t1.reshape(4 * _N, _GW)

  agg1a = segsum(t1f, src2_a, dst2)
  agg1b = segsum(t1f, src2_b, dst2)

  s1, st1 = pl.pallas_call(
      functools.partial(_sage_dense_body, self_is_pre=True),
      grid=(_GRID,),
      in_specs=[_row_spec(_H), _agg_spec(), _agg_spec(), _deg_spec(),
                _full_spec((_H, _H)), _full_spec((_H, _H)),
                _full_spec((1, _H))],
      out_specs=[_row_spec(_H), pl.BlockSpec((2, _H), lambda i: (0, 0))],
      out_shape=[jax.ShapeDtypeStruct((_N, _H), f32),
                 jax.ShapeDtypeStruct((2, _H), f32)],
  )(self1, agg1a, agg1b, deg3, W_self1, W_neigh1, b_sage1_2)

  out = pl.pallas_call(
      _tc3b_body,
      grid=(_GRID,),
      in_specs=[_row_spec(_H), _full_spec((2, _H)), _full_spec((1, _H)),
                _full_spec((1, _H)), _row_spec(_H),
                _full_spec((2 * _H, _H)), _full_spec((1, _H)),
                _full_spec((_H, _H // 2)), _full_spec((1, _H // 2)),
                _full_spec((_H // 2, 16)), _full_spec((1, 16))],
      out_specs=_row_spec(16),
      out_shape=jax.ShapeDtypeStruct((_N, 16), f32),
  )(s1, st1, gamma1_2, beta1_2, h0, W_rel, b_rel[None, :], W_c1,
    b_c1[None, :], W_c2, b_c2[None, :])

  return out


# speedup vs baseline: 6.3903x; 1.5140x over previous
import functools

import jax
import jax.numpy as jnp
from jax import lax
from jax.experimental import pallas as pl
from jax.experimental.pallas import tpu as pltpu
from jax.experimental.pallas import tpu_sc as plsc

_N = 50000
_H = 64
_NC = 2
_NS = 16
_CH = 128
_GW = 16
_ACC = 51200
_STRIPE = _ACC // _NS

_BN = 2000
_GRID = _N // _BN
_EPS = 1e-5

_SC_PARAMS = pltpu.CompilerParams(use_tc_tiling_on_sc=False)


_NB = 4
_LEAD = 2


def _make_segsum(cpt, gpt):
  mesh = plsc.VectorSubcoreMesh(core_axis_name="c", subcore_axis_name="s")
  ngrp = cpt // gpt
  scratch = [
      pltpu.VMEM((gpt, _CH), jnp.int32),
      pltpu.VMEM((gpt, _CH), jnp.int32),
      pltpu.VMEM((_CH, _GW), jnp.float32),
      pltpu.VMEM_SHARED((_ACC, _GW), jnp.float32),
  ]
  scratch += [pltpu.VMEM((_CH, _GW), jnp.float32) for _ in range(_NB)]
  scratch += [pltpu.SemaphoreType.DMA for _ in range(_NB)]

  def body(table, src4, dst2, agg_out, src_v, dst_v, zb, acc, *bufs):
    rows = bufs[:_NB]
    gsem = bufs[_NB:2 * _NB]
    c = lax.axis_index("c")
    s = lax.axis_index("s")
    base = s * _STRIPE

    @pl.loop(0, _CH)
    def _(r):
      zb[r, pl.ds(0, 16)] = jnp.zeros((16,), jnp.float32)

    def gather_wait(b):
      pltpu.make_async_copy(table.at[src_v.at[0]], rows[b], gsem[b]).wait()

    for p in range(2):
      @pl.loop(0, _STRIPE // _CH)
      def _(r):
        pltpu.sync_copy(zb, acc.at[pl.ds(base + r * _CH, _CH)])
      plsc.subcore_barrier()

      @pl.loop(0, ngrp)
      def _(g):
        row0 = s * cpt + g * gpt
        pltpu.sync_copy(src4.at[2 * p + c].at[pl.ds(row0, gpt)], src_v)
        pltpu.sync_copy(dst2.at[pl.ds(row0, gpt)], dst_v)

        for j in range(_LEAD):
          pltpu.async_copy(table.at[src_v.at[j]], rows[j], gsem[j])

        @pl.loop(0, gpt // _NB)
        def _(jj):
          for t in range(_NB):
            j = jj * _NB + t
            gather_wait(t)
            tn = (t + _LEAD) % _NB
            jn = j + _LEAD

            @pl.when(jn < gpt)
            def _():
              pltpu.async_copy(table.at[src_v.at[jn]], rows[tn], gsem[tn])
            pltpu.sync_copy(rows[t], acc.at[dst_v.at[j]], add=True)

      plsc.subcore_barrier()
      pltpu.sync_copy(acc.at[pl.ds(base, _STRIPE)],
                      agg_out.at[2 * p + c].at[pl.ds(base, _STRIPE)])

  return pl.kernel(
      body, out_type=jax.ShapeDtypeStruct((4, _ACC, _GW), jnp.float32),
      mesh=mesh, scratch_types=scratch, compiler_params=_SC_PARAMS)


def _make_deg(cpt, gpt):
  mesh = plsc.VectorSubcoreMesh(core_axis_name="c", subcore_axis_name="s")
  ngrp = cpt // gpt
  scratch = [
      pltpu.VMEM((gpt, _CH), jnp.int32),
      pltpu.VMEM((_CH,), jnp.float32),
      pltpu.VMEM((_CH,), jnp.float32),
      pltpu.VMEM_SHARED((_ACC,), jnp.float32),
  ]

  def body(dst2, deg_out, dst_v, ones, zrow, dacc):
    c = lax.axis_index("c")
    s = lax.axis_index("s")
    base = s * _STRIPE

    @pl.loop(0, _CH // 16)
    def _(r):
      ones[pl.ds(r * 16, 16)] = jnp.ones((16,), jnp.float32)
      zrow[pl.ds(r * 16, 16)] = jnp.zeros((16,), jnp.float32)

    @pl.loop(0, _STRIPE // _CH)
    def _(r):
      pltpu.sync_copy(zrow, dacc.at[pl.ds(base + r * _CH, _CH)])
    plsc.subcore_barrier()

    @pl.loop(0, ngrp)
    def _(g):
      pltpu.sync_copy(dst2.at[pl.ds(s * cpt + g * gpt, gpt)], dst_v)

      @pl.loop(0, gpt)
      def _(j):
        @pl.when((j % 2) == c)
        def _():
          pltpu.sync_copy(ones, dacc.at[dst_v.at[j]], add=True)

    plsc.subcore_barrier()
    pltpu.sync_copy(dacc.at[pl.ds(base, _STRIPE)],
                    deg_out.at[c].at[pl.ds(base, _STRIPE)])

  return pl.kernel(
      body, out_type=jax.ShapeDtypeStruct((_NC, _ACC), jnp.float32),
      mesh=mesh, scratch_types=scratch, compiler_params=_SC_PARAMS)


def _tc1_body(s_ref, m_ref, w_ref, b_ref, h0_ref, t0_ref):
  w = w_ref[...]
  h = jnp.dot(s_ref[...], w[:_H], preferred_element_type=jnp.float32)
  h += jnp.dot(m_ref[...], w[_H:], preferred_element_type=jnp.float32)
  h = jnp.maximum(h + b_ref[...], 0.0)
  h0_ref[...] = h
  t0_ref[...] = jnp.stack([h[:, 0:16], h[:, 16:32], h[:, 32:48], h[:, 48:64]],
                          axis=0)


def _sage_dense_body(hself_ref, aa_ref, d_ref, ws_ref, wn_ref, b_ref,
                     s_out_ref, st_ref, *, self_is_pre):
  i = pl.program_id(0)
  aa = aa_ref[...]
  d = d_ref[...]
  deg = jnp.maximum(d[0] + d[1], 1.0)
  hn = jnp.concatenate([aa[0], aa[1], aa[2], aa[3]], axis=1) / deg
  if self_is_pre:
    s = hself_ref[...] + hn + b_ref[...]
  else:
    s = jnp.dot(hself_ref[...], ws_ref[...],
                preferred_element_type=jnp.float32)
    s += jnp.dot(hn, wn_ref[...], preferred_element_type=jnp.float32)
    s += b_ref[...]
  s_out_ref[...] = s

  @pl.when(i == 0)
  def _():
    st_ref[...] = jnp.zeros_like(st_ref)
  st_ref[...] += jnp.stack([jnp.sum(s, axis=0), jnp.sum(s * s, axis=0)])


def _tc2b_body(s_ref, st_ref, g_ref, be_ref, ws_ref, wn_ref,
               self1_ref, t1_ref):
  st = st_ref[...]
  mean = st[0:1] / _N
  var = st[1:2] / _N - mean * mean
  inv = lax.rsqrt(var + _EPS)
  h1 = jnp.maximum((s_ref[...] - mean) * inv * g_ref[...] + be_ref[...], 0.0)
  self1_ref[...] = jnp.dot(h1, ws_ref[...],
                           preferred_element_type=jnp.float32)
  p1 = jnp.dot(h1, wn_ref[...], preferred_element_type=jnp.float32)
  t1_ref[...] = jnp.stack(
      [p1[:, 0:16], p1[:, 16:32], p1[:, 32:48], p1[:, 48:64]], axis=0)


def _tc3b_body(s_ref, st_ref, g_ref, be_ref, h0_ref, wrel_ref, brel_ref,
               wc1_ref, bc1_ref, wc2_ref, bc2_ref, out_ref):
  st = st_ref[...]
  mean = st[0:1] / _N
  var = st[1:2] / _N - mean * mean
  inv = lax.rsqrt(var + _EPS)
  h2 = jnp.maximum((s_ref[...] - mean) * inv * g_ref[...] + be_ref[...], 0.0)
  wrel = wrel_ref[...]
  hf = jnp.dot(h0_ref[...], wrel[:_H], preferred_element_type=jnp.float32)
  hf += jnp.dot(h2, wrel[_H:], preferred_element_type=jnp.float32)
  hf = jnp.maximum(hf + brel_ref[...], 0.0)
  hid = jnp.maximum(
      jnp.dot(hf, wc1_ref[...], preferred_element_type=jnp.float32)
      + bc1_ref[...], 0.0)
  out_ref[...] = (jnp.dot(hid, wc2_ref[...],
                          preferred_element_type=jnp.float32) + bc2_ref[...])


def _row_spec(width):
  return pl.BlockSpec((_BN, width), lambda i: (i, 0))


def _full_spec(shape):
  nd = len(shape)
  return pl.BlockSpec(shape, lambda i, _n=nd: (0,) * _n)


def _agg_spec():
  return pl.BlockSpec((4, _BN, _GW), lambda i: (0, i, 0))


def _deg_spec():
  return pl.BlockSpec((_NC, _BN, 1), lambda i: (0, i, 0))


def _table_spec():
  return pl.BlockSpec((4, _BN, _GW), lambda i: (0, i, 0))


def kernel(structural_features, multimodal_features, edge_index, W_in, b_in,
           W_self0, W_neigh0, b_sage0, gamma0, beta0, W_self1, W_neigh1,
           b_sage1, gamma1, beta1, W_rel, b_rel, W_c1, b_c1, W_c2, b_c2):
  f32 = jnp.float32
  src = edge_index[0]
  dst = edge_index[1]
  e = src.shape[0]

  gpt = 56
  cpt = -(-e // (_CH * _NS))
  cpt = -(-cpt // gpt) * gpt
  e_pad = cpt * _CH * _NS
  padn = e_pad - e
  srcp = jnp.concatenate([src, jnp.zeros((padn,), jnp.int32)])
  dstp = jnp.concatenate([dst, jnp.full((padn,), _N, jnp.int32)])
  src4 = (srcp[None, :] + (_N * jnp.arange(4, dtype=jnp.int32))[:, None]
          ).reshape(4, e_pad // _CH, _CH)
  dst2 = dstp.reshape(e_pad // _CH, _CH)

  segsum = _make_segsum(cpt, gpt)
  degk = _make_deg(cpt, gpt)

  b_in2 = b_in[None, :]
  b_sage0_2 = b_sage0[None, :]
  b_sage1_2 = b_sage1[None, :]
  gamma0_2, beta0_2 = gamma0[None, :], beta0[None, :]
  gamma1_2, beta1_2 = gamma1[None, :], beta1[None, :]

  degp = degk(dst2)
  deg3 = degp[:, :, None]

  h0, t0 = pl.pallas_call(
      _tc1_body,
      grid=(_GRID,),
      in_specs=[_row_spec(_H), _row_spec(_H), _full_spec((2 * _H, _H)),
                _full_spec((1, _H))],
      out_specs=[_row_spec(_H), _table_spec()],
      out_shape=[jax.ShapeDtypeStruct((_N, _H), f32),
                 jax.ShapeDtypeStruct((4, _N, _GW), f32)],
  )(structural_features, multimodal_features, W_in, b_in2)
  t0f = t0.reshape(4 * _N, _GW)

  agg0 = segsum(t0f, src4, dst2)

  s0, st0 = pl.pallas_call(
      functools.partial(_sage_dense_body, self_is_pre=False),
      grid=(_GRID,),
      in_specs=[_row_spec(_H), _agg_spec(), _deg_spec(),
                _full_spec((_H, 2 * _H)), _full_spec((_H, 2 * _H)),
                _full_spec((1, 2 * _H))],
      out_specs=[_row_spec(2 * _H),
                 pl.BlockSpec((2, 2 * _H), lambda i: (0, 0))],
      out_shape=[jax.ShapeDtypeStruct((_N, 2 * _H), f32),
                 jax.ShapeDtypeStruct((2, 2 * _H), f32)],
  )(h0, agg0, deg3, W_self0, W_neigh0, b_sage0_2)

  self1, t1 = pl.pallas_call(
      _tc2b_body,
      grid=(_GRID,),
      in_specs=[_row_spec(2 * _H), _full_spec((2, 2 * _H)),
                _full_spec((1, 2 * _H)), _full_spec((1, 2 * _H)),
                _full_spec((2 * _H, _H)), _full_spec((2 * _H, _H))],
      out_specs=[_row_spec(_H), _table_spec()],
      out_shape=[jax.ShapeDtypeStruct((_N, _H), f32),
                 jax.ShapeDtypeStruct((4, _N, _GW), f32)],
  )(s0, st0, gamma0_2, beta0_2, W_self1, W_neigh1)
  t1f = t1.reshape(4 * _N, _GW)

  agg1 = segsum(t1f, src4, dst2)

  s1, st1 = pl.pallas_call(
      functools.partial(_sage_dense_body, self_is_pre=True),
      grid=(_GRID,),
      in_specs=[_row_spec(_H), _agg_spec(), _deg_spec(),
                _full_spec((_H, _H)), _full_spec((_H, _H)),
                _full_spec((1, _H))],
      out_specs=[_row_spec(_H), pl.BlockSpec((2, _H), lambda i: (0, 0))],
      out_shape=[jax.ShapeDtypeStruct((_N, _H), f32),
                 jax.ShapeDtypeStruct((2, _H), f32)],
  )(self1, agg1, deg3, W_self1, W_neigh1, b_sage1_2)

  out = pl.pallas_call(
      _tc3b_body,
      grid=(_GRID,),
      in_specs=[_row_spec(_H), _full_spec((2, _H)), _full_spec((1, _H)),
                _full_spec((1, _H)), _row_spec(_H),
                _full_spec((2 * _H, _H)), _full_spec((1, _H)),
                _full_spec((_H, _H // 2)), _full_spec((1, _H // 2)),
                _full_spec((_H // 2, 16)), _full_spec((1, 16))],
      out_specs=_row_spec(16),
      out_shape=jax.ShapeDtypeStruct((_N, 16), f32),
  )(s1, st1, gamma1_2, beta1_2, h0, W_rel, b_rel[None, :], W_c1,
    b_c1[None, :], W_c2, b_c2[None, :])

  return out


# speedup vs baseline: 8.2131x; 1.2853x over previous
import functools

import jax
import jax.numpy as jnp
from jax import lax
from jax.experimental import pallas as pl
from jax.experimental.pallas import tpu as pltpu
from jax.experimental.pallas import tpu_sc as plsc

_N = 50000
_H = 64
_NC = 2
_NS = 16
_CH = 128
_GW = 16
_ACC = 51200
_STRIPE = _ACC // _NS
_NP = _ACC // 2

_BP = 1600
_GRID = _NP // _BP
_EPS = 1e-5

_SC_PARAMS = pltpu.CompilerParams(use_tc_tiling_on_sc=False)

_NB = 4
_LEAD = 2


def _make_segsum(cpt, gpt):
  mesh = plsc.VectorSubcoreMesh(core_axis_name="c", subcore_axis_name="s")
  ngrp = cpt // gpt
  scratch = [
      pltpu.VMEM((gpt, _CH), jnp.int32),
      pltpu.VMEM((gpt, _CH), jnp.int32),
      pltpu.VMEM((_CH, _GW), jnp.float32),
      pltpu.VMEM_SHARED((_ACC, _GW), jnp.float32),
  ]
  scratch += [pltpu.VMEM((_CH, _GW), jnp.float32) for _ in range(_NB)]
  scratch += [pltpu.SemaphoreType.DMA for _ in range(_NB)]

  def body(table, src4, dst2, agg_out, src_v, dst_v, zb, acc, *bufs):
    rows = bufs[:_NB]
    gsem = bufs[_NB:2 * _NB]
    c = lax.axis_index("c")
    s = lax.axis_index("s")
    base = s * _STRIPE

    @pl.loop(0, _CH)
    def _(r):
      zb[r, pl.ds(0, 16)] = jnp.zeros((16,), jnp.float32)

    def gather_wait(b):
      pltpu.make_async_copy(table.at[src_v.at[0]], rows[b], gsem[b]).wait()

    for p in range(2):
      @pl.loop(0, _STRIPE // _CH)
      def _(r):
        pltpu.sync_copy(zb, acc.at[pl.ds(base + r * _CH, _CH)])
      plsc.subcore_barrier()

      @pl.loop(0, ngrp)
      def _(g):
        row0 = s * cpt + g * gpt
        pltpu.sync_copy(src4.at[2 * p + c].at[pl.ds(row0, gpt)], src_v)
        pltpu.sync_copy(dst2.at[pl.ds(row0, gpt)], dst_v)

        for j in range(_LEAD):
          pltpu.async_copy(table.at[src_v.at[j]], rows[j], gsem[j])

        @pl.loop(0, gpt // _NB)
        def _(jj):
          for t in range(_NB):
            j = jj * _NB + t
            gather_wait(t)
            tn = (t + _LEAD) % _NB
            jn = j + _LEAD

            @pl.when(jn < gpt)
            def _():
              pltpu.async_copy(table.at[src_v.at[jn]], rows[tn], gsem[tn])
            pltpu.sync_copy(rows[t], acc.at[dst_v.at[j]], add=True)

      plsc.subcore_barrier()
      pltpu.sync_copy(acc.at[pl.ds(base, _STRIPE)],
                      agg_out.at[pl.ds(base, _STRIPE), 2 * p + c])

  return pl.kernel(
      body, out_type=jax.ShapeDtypeStruct((_ACC, 4, _GW), jnp.float32),
      mesh=mesh, scratch_types=scratch, compiler_params=_SC_PARAMS)


def _make_deg(cpt, gpt):
  mesh = plsc.VectorSubcoreMesh(core_axis_name="c", subcore_axis_name="s")
  ngrp = cpt // gpt
  scratch = [
      pltpu.VMEM((gpt, _CH), jnp.int32),
      pltpu.VMEM((_CH,), jnp.float32),
      pltpu.VMEM((_CH,), jnp.float32),
      pltpu.VMEM_SHARED((_ACC,), jnp.float32),
  ]

  def body(dst2, deg_out, dst_v, ones, zrow, dacc):
    c = lax.axis_index("c")
    s = lax.axis_index("s")
    base = s * _STRIPE

    @pl.loop(0, _CH // 16)
    def _(r):
      ones[pl.ds(r * 16, 16)] = jnp.ones((16,), jnp.float32)
      zrow[pl.ds(r * 16, 16)] = jnp.zeros((16,), jnp.float32)

    @pl.loop(0, _STRIPE // _CH)
    def _(r):
      pltpu.sync_copy(zrow, dacc.at[pl.ds(base + r * _CH, _CH)])
    plsc.subcore_barrier()

    @pl.loop(0, ngrp)
    def _(g):
      pltpu.sync_copy(dst2.at[pl.ds(s * cpt + g * gpt, gpt)], dst_v)

      @pl.loop(0, gpt)
      def _(j):
        @pl.when((j % 2) == c)
        def _():
          pltpu.sync_copy(ones, dacc.at[dst_v.at[j]], add=True)

    plsc.subcore_barrier()
    pltpu.sync_copy(dacc.at[pl.ds(base, _STRIPE)],
                    deg_out.at[c].at[pl.ds(base, _STRIPE)])

  return pl.kernel(
      body, out_type=jax.ShapeDtypeStruct((_NC, _ACC), jnp.float32),
      mesh=mesh, scratch_types=scratch, compiler_params=_SC_PARAMS)


def _halves(x):
  return x[:, :_H], x[:, _H:]


def _dot(a, b):
  return jnp.dot(a, b, preferred_element_type=jnp.float32)


def _tc1_body(s_ref, m_ref, w_ref, b_ref, h0_ref):
  w = w_ref[...]
  ws, wm = w[:_H], w[_H:]
  b = b_ref[...]
  se, so = _halves(s_ref[...])
  me, mo = _halves(m_ref[...])
  he = jnp.maximum(_dot(se, ws) + _dot(me, wm) + b, 0.0)
  ho = jnp.maximum(_dot(so, ws) + _dot(mo, wm) + b, 0.0)
  h0_ref[...] = jnp.concatenate([he, ho], axis=1)


def _mask(i):
  rid = i * _BP + lax.broadcasted_iota(jnp.int32, (_BP, 1), 0)
  return rid < (_N // 2)


def _stats_update(i, st_ref, se, so):
  m = _mask(i)
  sem = jnp.where(m, se, 0.0)
  som = jnp.where(m, so, 0.0)

  @pl.when(i == 0)
  def _():
    st_ref[...] = jnp.zeros_like(st_ref)
  st_ref[...] += jnp.stack([
      jnp.sum(sem, axis=0) + jnp.sum(som, axis=0),
      jnp.sum(sem * sem, axis=0) + jnp.sum(som * som, axis=0)])


def _sage0_body(h0_ref, a_ref, rp_ref, ws_ref, wn_ref, b_ref,
                s_out_ref, st_ref):
  i = pl.program_id(0)
  hn = a_ref[...] * rp_ref[...]
  ae, ao = _halves(hn)
  h0e, h0o = _halves(h0_ref[...])
  ws, wn, b = ws_ref[...], wn_ref[...], b_ref[...]
  se = _dot(h0e, ws) + _dot(ae, wn) + b
  so = _dot(h0o, ws) + _dot(ao, wn) + b
  s_out_ref[...] = jnp.concatenate([se, so], axis=1)
  _stats_update(i, st_ref, se, so)


def _bnorm(x, st, g, be):
  mean = st[0:1] / _N
  var = st[1:2] / _N - mean * mean
  inv = lax.rsqrt(var + _EPS)
  return jnp.maximum((x - mean) * inv * g + be, 0.0)


def _tc2b_body(s_ref, st_ref, g_ref, be_ref, ws_ref, wn_ref,
               self1_ref, t1_ref):
  st, g, be = st_ref[...], g_ref[...], be_ref[...]
  ws, wn = ws_ref[...], wn_ref[...]
  s = s_ref[...]
  h1e = _bnorm(s[:, :2 * _H], st, g, be)
  h1o = _bnorm(s[:, 2 * _H:], st, g, be)
  self1_ref[...] = jnp.concatenate([_dot(h1e, ws), _dot(h1o, ws)], axis=1)
  t1_ref[...] = jnp.concatenate([_dot(h1e, wn), _dot(h1o, wn)], axis=1)


def _sage1_body(self1_ref, a_ref, rp_ref, b_ref, s_out_ref, st_ref):
  i = pl.program_id(0)
  b = b_ref[...]
  s = (self1_ref[...] + a_ref[...] * rp_ref[...]
       + jnp.concatenate([b, b], axis=1))
  s_out_ref[...] = s
  se, so = _halves(s)
  _stats_update(i, st_ref, se, so)


def _tc3b_body(s_ref, st_ref, g_ref, be_ref, h0_ref, wrel_ref, brel_ref,
               wc1_ref, bc1_ref, wc2_ref, bc2_ref, out_ref):
  st, g, be = st_ref[...], g_ref[...], be_ref[...]
  wrel, brel = wrel_ref[...], brel_ref[...]
  wra, wrb = wrel[:_H], wrel[_H:]
  wc1, bc1 = wc1_ref[...], bc1_ref[...]
  wc2, bc2 = wc2_ref[...], bc2_ref[...]
  s = s_ref[...]
  h0e, h0o = _halves(h0_ref[...])

  def head(h0h, sh):
    h2 = _bnorm(sh, st, g, be)
    hf = jnp.maximum(_dot(h0h, wra) + _dot(h2, wrb) + brel, 0.0)
    hid = jnp.maximum(_dot(hf, wc1) + bc1, 0.0)
    return _dot(hid, wc2) + bc2

  oe = head(h0e, s[:, :_H])
  oo = head(h0o, s[:, _H:])
  out_ref[...] = jnp.concatenate([oe, oo], axis=1)


def _row_spec(width):
  return pl.BlockSpec((_BP, width), lambda i: (i, 0))


def _full_spec(shape):
  nd = len(shape)
  return pl.BlockSpec(shape, lambda i, _n=nd: (0,) * _n)


def kernel(structural_features, multimodal_features, edge_index, W_in, b_in,
           W_self0, W_neigh0, b_sage0, gamma0, beta0, W_self1, W_neigh1,
           b_sage1, gamma1, beta1, W_rel, b_rel, W_c1, b_c1, W_c2, b_c2):
  f32 = jnp.float32
  src = edge_index[0]
  dst = edge_index[1]
  e = src.shape[0]

  gpt = 56
  cpt = -(-e // (_CH * _NS))
  cpt = -(-cpt // gpt) * gpt
  e_pad = cpt * _CH * _NS
  padn = e_pad - e
  srcp = jnp.concatenate([src, jnp.zeros((padn,), jnp.int32)])
  dstp = jnp.concatenate([dst, jnp.full((padn,), _N, jnp.int32)])
  src4 = (4 * srcp[None, :] + jnp.arange(4, dtype=jnp.int32)[:, None]
          ).reshape(4, e_pad // _CH, _CH)
  dst2 = dstp.reshape(e_pad // _CH, _CH)

  segsum = _make_segsum(cpt, gpt)
  degk = _make_deg(cpt, gpt)

  s128 = structural_features.reshape(_N // 2, 2 * _H)
  m128 = multimodal_features.reshape(_N // 2, 2 * _H)

  b_in2 = b_in[None, :]
  b_sage0_2 = b_sage0[None, :]
  b_sage1_2 = b_sage1[None, :]
  gamma0_2, beta0_2 = gamma0[None, :], beta0[None, :]
  gamma1_2, beta1_2 = gamma1[None, :], beta1[None, :]

  degp = degk(dst2)
  rpack = jnp.broadcast_to(
      (1.0 / jnp.maximum(degp[0] + degp[1], 1.0))[:, None],
      (_ACC, _H)).reshape(_NP, 2 * _H)

  h0p = pl.pallas_call(
      _tc1_body,
      grid=(_GRID,),
      in_specs=[_row_spec(2 * _H), _row_spec(2 * _H),
                _full_spec((2 * _H, _H)), _full_spec((1, _H))],
      out_specs=_row_spec(2 * _H),
      out_shape=jax.ShapeDtypeStruct((_NP, 2 * _H), f32),
  )(s128, m128, W_in, b_in2)

  agg0 = segsum(h0p.reshape(4 * _ACC, _GW), src4, dst2)
  agg0p = agg0.reshape(_NP, 2 * _H)

  s0p, st0 = pl.pallas_call(
      _sage0_body,
      grid=(_GRID,),
      in_specs=[_row_spec(2 * _H), _row_spec(2 * _H), _row_spec(2 * _H),
                _full_spec((_H, 2 * _H)), _full_spec((_H, 2 * _H)),
                _full_spec((1, 2 * _H))],
      out_specs=[_row_spec(4 * _H),
                 pl.BlockSpec((2, 2 * _H), lambda i: (0, 0))],
      out_shape=[jax.ShapeDtypeStruct((_NP, 4 * _H), f32),
                 jax.ShapeDtypeStruct((2, 2 * _H), f32)],
  )(h0p, agg0p, rpack, W_self0, W_neigh0, b_sage0_2)

  self1p, t1p = pl.pallas_call(
      _tc2b_body,
      grid=(_GRID,),
      in_specs=[_row_spec(4 * _H), _full_spec((2, 2 * _H)),
                _full_spec((1, 2 * _H)), _full_spec((1, 2 * _H)),
                _full_spec((2 * _H, _H)), _full_spec((2 * _H, _H))],
      out_specs=[_row_spec(2 * _H), _row_spec(2 * _H)],
      out_shape=[jax.ShapeDtypeStruct((_NP, 2 * _H), f32),
                 jax.ShapeDtypeStruct((_NP, 2 * _H), f32)],
  )(s0p, st0, gamma0_2, beta0_2, W_self1, W_neigh1)

  agg1 = segsum(t1p.reshape(4 * _ACC, _GW), src4, dst2)
  agg1p = agg1.reshape(_NP, 2 * _H)

  s1p, st1 = pl.pallas_call(
      _sage1_body,
      grid=(_GRID,),
      in_specs=[_row_spec(2 * _H), _row_spec(2 * _H), _row_spec(2 * _H),
                _full_spec((1, _H))],
      out_specs=[_row_spec(2 * _H), pl.BlockSpec((2, _H), lambda i: (0, 0))],
      out_shape=[jax.ShapeDtypeStruct((_NP, 2 * _H), f32),
                 jax.ShapeDtypeStruct((2, _H), f32)],
  )(self1p, agg1p, rpack, b_sage1_2)

  outp = pl.pallas_call(
      _tc3b_body,
      grid=(_GRID,),
      in_specs=[_row_spec(2 * _H), _full_spec((2, _H)), _full_spec((1, _H)),
                _full_spec((1, _H)), _row_spec(2 * _H),
                _full_spec((2 * _H, _H)), _full_spec((1, _H)),
                _full_spec((_H, _H // 2)), _full_spec((1, _H // 2)),
                _full_spec((_H // 2, 16)), _full_spec((1, 16))],
      out_specs=_row_spec(32),
      out_shape=jax.ShapeDtypeStruct((_NP, 32), f32),
  )(s1p, st1, gamma1_2, beta1_2, h0p, W_rel, b_rel[None, :], W_c1,
    b_c1[None, :], W_c2, b_c2[None, :])

  return outp.reshape(2 * _NP, 16)[:_N]


# speedup vs baseline: 9.6228x; 1.1716x over previous
import functools

import jax
import jax.numpy as jnp
from jax import lax
from jax.experimental import pallas as pl
from jax.experimental.pallas import tpu as pltpu
from jax.experimental.pallas import tpu_sc as plsc

_N = 50000
_H = 64
_NC = 2
_NS = 16
_CH = 128
_GW = 16
_ACC = 51200
_STRIPE = _ACC // _NS
_NP = _ACC // 2

_BP = 1600
_GRID = _NP // _BP
_EPS = 1e-5

_SC_PARAMS = pltpu.CompilerParams(use_tc_tiling_on_sc=False)

_NB = 7
_LEAD = 3


def _make_segsum(cpt, gpt):
  mesh = plsc.VectorSubcoreMesh(core_axis_name="c", subcore_axis_name="s")
  ngrp = cpt // gpt
  scratch = [
      pltpu.VMEM((gpt, _CH), jnp.int32),
      pltpu.VMEM((gpt, _CH), jnp.int32),
      pltpu.VMEM((_CH, _GW), jnp.float32),
      pltpu.VMEM_SHARED((_ACC, _GW), jnp.float32),
  ]
  scratch += [pltpu.VMEM((_CH, _GW), jnp.float32) for _ in range(_NB)]
  scratch += [pltpu.SemaphoreType.DMA for _ in range(2 * _NB)]

  def body(table, src4, dst2, agg_out, src_v, dst_v, zb, acc, *bufs):
    rows = bufs[:_NB]
    gsem = bufs[_NB:2 * _NB]
    ssem = bufs[2 * _NB:]
    c = lax.axis_index("c")
    s = lax.axis_index("s")
    base = s * _STRIPE

    @pl.loop(0, _CH)
    def _(r):
      zb[r, pl.ds(0, 16)] = jnp.zeros((16,), jnp.float32)

    def gather_wait(b):
      pltpu.make_async_copy(table.at[src_v.at[0]], rows[b], gsem[b]).wait()

    def scatter_wait(b):
      pltpu.make_async_copy(rows[b], acc.at[dst_v.at[0]], ssem[b]).wait()

    for p in range(2):
      @pl.loop(0, _STRIPE // _CH)
      def _(r):
        pltpu.sync_copy(zb, acc.at[pl.ds(base + r * _CH, _CH)])
      plsc.subcore_barrier()

      @pl.loop(0, ngrp)
      def _(g):
        row0 = s * cpt + g * gpt
        pltpu.sync_copy(src4.at[2 * p + c].at[pl.ds(row0, gpt)], src_v)
        pltpu.sync_copy(dst2.at[pl.ds(row0, gpt)], dst_v)

        for j in range(_LEAD):
          pltpu.async_copy(table.at[src_v.at[j]], rows[j], gsem[j])

        @pl.loop(0, gpt // _NB)
        def _(jj):
          for t in range(_NB):
            j = jj * _NB + t
            gather_wait(t)
            pltpu.async_copy(rows[t], acc.at[dst_v.at[j]], ssem[t],
                             add=True)
            tn = (t + _LEAD) % _NB
            jn = j + _LEAD

            @pl.when(jn < gpt)
            def _():
              @pl.when(jn >= _NB)
              def _():
                scatter_wait(tn)
              pltpu.async_copy(table.at[src_v.at[jn]], rows[tn], gsem[tn])

        for b in range(_NB):
          scatter_wait(b)

      plsc.subcore_barrier()
      pltpu.sync_copy(acc.at[pl.ds(base, _STRIPE)],
                      agg_out.at[pl.ds(base, _STRIPE), 2 * p + c])

  return pl.kernel(
      body, out_type=jax.ShapeDtypeStruct((_ACC, 4, _GW), jnp.float32),
      mesh=mesh, scratch_types=scratch, compiler_params=_SC_PARAMS)


def _make_deg(cpt, gpt):
  mesh = plsc.VectorSubcoreMesh(core_axis_name="c", subcore_axis_name="s")
  ngrp = cpt // gpt
  scratch = [
      pltpu.VMEM((gpt, _CH), jnp.int32),
      pltpu.VMEM((_CH,), jnp.float32),
      pltpu.VMEM((_CH,), jnp.float32),
      pltpu.VMEM_SHARED((_ACC,), jnp.float32),
  ]

  def body(dst2, deg_out, dst_v, ones, zrow, dacc):
    c = lax.axis_index("c")
    s = lax.axis_index("s")
    base = s * _STRIPE

    @pl.loop(0, _CH // 16)
    def _(r):
      ones[pl.ds(r * 16, 16)] = jnp.ones((16,), jnp.float32)
      zrow[pl.ds(r * 16, 16)] = jnp.zeros((16,), jnp.float32)

    @pl.loop(0, _STRIPE // _CH)
    def _(r):
      pltpu.sync_copy(zrow, dacc.at[pl.ds(base + r * _CH, _CH)])
    plsc.subcore_barrier()

    @pl.loop(0, ngrp)
    def _(g):
      pltpu.sync_copy(dst2.at[pl.ds(s * cpt + g * gpt, gpt)], dst_v)

      @pl.loop(0, gpt)
      def _(j):
        @pl.when((j % 2) == c)
        def _():
          pltpu.sync_copy(ones, dacc.at[dst_v.at[j]], add=True)

    plsc.subcore_barrier()
    pltpu.sync_copy(dacc.at[pl.ds(base, _STRIPE)],
                    deg_out.at[c].at[pl.ds(base, _STRIPE)])

  return pl.kernel(
      body, out_type=jax.ShapeDtypeStruct((_NC, _ACC), jnp.float32),
      mesh=mesh, scratch_types=scratch, compiler_params=_SC_PARAMS)


def _halves(x):
  return x[:, :_H], x[:, _H:]


def _dot(a, b):
  return jnp.dot(a, b, preferred_element_type=jnp.float32)


def _tc1_body(s_ref, m_ref, w_ref, b_ref, h0_ref):
  w = w_ref[...]
  ws, wm = w[:_H], w[_H:]
  b = b_ref[...]
  se, so = _halves(s_ref[...])
  me, mo = _halves(m_ref[...])
  he = jnp.maximum(_dot(se, ws) + _dot(me, wm) + b, 0.0)
  ho = jnp.maximum(_dot(so, ws) + _dot(mo, wm) + b, 0.0)
  h0_ref[...] = jnp.concatenate([he, ho], axis=1)


def _mask(i):
  rid = i * _BP + lax.broadcasted_iota(jnp.int32, (_BP, 1), 0)
  return rid < (_N // 2)


def _stats_update(i, st_ref, se, so):
  m = _mask(i)
  sem = jnp.where(m, se, 0.0)
  som = jnp.where(m, so, 0.0)

  @pl.when(i == 0)
  def _():
    st_ref[...] = jnp.zeros_like(st_ref)
  st_ref[...] += jnp.stack([
      jnp.sum(sem, axis=0) + jnp.sum(som, axis=0),
      jnp.sum(sem * sem, axis=0) + jnp.sum(som * som, axis=0)])


def _sage0_body(h0_ref, a_ref, rp_ref, ws_ref, wn_ref, b_ref,
                s_out_ref, st_ref):
  i = pl.program_id(0)
  hn = a_ref[...] * rp_ref[...]
  ae, ao = _halves(hn)
  h0e, h0o = _halves(h0_ref[...])
  ws, wn, b = ws_ref[...], wn_ref[...], b_ref[...]
  se = _dot(h0e, ws) + _dot(ae, wn) + b
  so = _dot(h0o, ws) + _dot(ao, wn) + b
  s_out_ref[...] = jnp.concatenate([se, so], axis=1)
  _stats_update(i, st_ref, se, so)


def _bnorm(x, st, g, be):
  mean = st[0:1] / _N
  var = st[1:2] / _N - mean * mean
  inv = lax.rsqrt(var + _EPS)
  return jnp.maximum((x - mean) * inv * g + be, 0.0)


def _tc2b_body(s_ref, st_ref, g_ref, be_ref, ws_ref, wn_ref,
               self1_ref, t1_ref):
  st, g, be = st_ref[...], g_ref[...], be_ref[...]
  ws, wn = ws_ref[...], wn_ref[...]
  s = s_ref[...]
  h1e = _bnorm(s[:, :2 * _H], st, g, be)
  h1o = _bnorm(s[:, 2 * _H:], st, g, be)
  self1_ref[...] = jnp.concatenate([_dot(h1e, ws), _dot(h1o, ws)], axis=1)
  t1_ref[...] = jnp.concatenate([_dot(h1e, wn), _dot(h1o, wn)], axis=1)


def _sage1_body(self1_ref, a_ref, rp_ref, b_ref, s_out_ref, st_ref):
  i = pl.program_id(0)
  b = b_ref[...]
  s = (self1_ref[...] + a_ref[...] * rp_ref[...]
       + jnp.concatenate([b, b], axis=1))
  s_out_ref[...] = s
  se, so = _halves(s)
  _stats_update(i, st_ref, se, so)


def _tc3b_body(s_ref, st_ref, g_ref, be_ref, h0_ref, wrel_ref, brel_ref,
               wc1_ref, bc1_ref, wc2_ref, bc2_ref, out_ref):
  st, g, be = st_ref[...], g_ref[...], be_ref[...]
  wrel, brel = wrel_ref[...], brel_ref[...]
  wra, wrb = wrel[:_H], wrel[_H:]
  wc1, bc1 = wc1_ref[...], bc1_ref[...]
  wc2, bc2 = wc2_ref[...], bc2_ref[...]
  s = s_ref[...]
  h0e, h0o = _halves(h0_ref[...])

  def head(h0h, sh):
    h2 = _bnorm(sh, st, g, be)
    hf = jnp.maximum(_dot(h0h, wra) + _dot(h2, wrb) + brel, 0.0)
    hid = jnp.maximum(_dot(hf, wc1) + bc1, 0.0)
    return _dot(hid, wc2) + bc2

  oe = head(h0e, s[:, :_H])
  oo = head(h0o, s[:, _H:])
  out_ref[...] = jnp.concatenate([oe, oo], axis=1)


def _row_spec(width):
  return pl.BlockSpec((_BP, width), lambda i: (i, 0))


def _full_spec(shape):
  nd = len(shape)
  return pl.BlockSpec(shape, lambda i, _n=nd: (0,) * _n)


def kernel(structural_features, multimodal_features, edge_index, W_in, b_in,
           W_self0, W_neigh0, b_sage0, gamma0, beta0, W_self1, W_neigh1,
           b_sage1, gamma1, beta1, W_rel, b_rel, W_c1, b_c1, W_c2, b_c2):
  f32 = jnp.float32
  src = edge_index[0]
  dst = edge_index[1]
  e = src.shape[0]

  gpt = 56
  cpt = -(-e // (_CH * _NS))
  cpt = -(-cpt // gpt) * gpt
  e_pad = cpt * _CH * _NS
  padn = e_pad - e
  srcp = jnp.concatenate([src, jnp.zeros((padn,), jnp.int32)])
  dstp = jnp.concatenate([dst, jnp.full((padn,), _N, jnp.int32)])
  src4 = (4 * srcp[None, :] + jnp.arange(4, dtype=jnp.int32)[:, None]
          ).reshape(4, e_pad // _CH, _CH)
  dst2 = dstp.reshape(e_pad // _CH, _CH)

  segsum = _make_segsum(cpt, gpt)
  degk = _make_deg(cpt, gpt)

  s128 = structural_features.reshape(_N // 2, 2 * _H)
  m128 = multimodal_features.reshape(_N // 2, 2 * _H)

  b_in2 = b_in[None, :]
  b_sage0_2 = b_sage0[None, :]
  b_sage1_2 = b_sage1[None, :]
  gamma0_2, beta0_2 = gamma0[None, :], beta0[None, :]
  gamma1_2, beta1_2 = gamma1[None, :], beta1[None, :]

  degp = degk(dst2)
  rpack = jnp.broadcast_to(
      (1.0 / jnp.maximum(degp[0] + degp[1], 1.0))[:, None],
      (_ACC, _H)).reshape(_NP, 2 * _H)

  h0p = pl.pallas_call(
      _tc1_body,
      grid=(_GRID,),
      in_specs=[_row_spec(2 * _H), _row_spec(2 * _H),
                _full_spec((2 * _H, _H)), _full_spec((1, _H))],
      out_specs=_row_spec(2 * _H),
      out_shape=jax.ShapeDtypeStruct((_NP, 2 * _H), f32),
  )(s128, m128, W_in, b_in2)

  agg0 = segsum(h0p.reshape(4 * _ACC, _GW), src4, dst2)
  agg0p = agg0.reshape(_NP, 2 * _H)

  s0p, st0 = pl.pallas_call(
      _sage0_body,
      grid=(_GRID,),
      in_specs=[_row_spec(2 * _H), _row_spec(2 * _H), _row_spec(2 * _H),
                _full_spec((_H, 2 * _H)), _full_spec((_H, 2 * _H)),
                _full_spec((1, 2 * _H))],
      out_specs=[_row_spec(4 * _H),
                 pl.BlockSpec((2, 2 * _H), lambda i: (0, 0))],
      out_shape=[jax.ShapeDtypeStruct((_NP, 4 * _H), f32),
                 jax.ShapeDtypeStruct((2, 2 * _H), f32)],
  )(h0p, agg0p, rpack, W_self0, W_neigh0, b_sage0_2)

  self1p, t1p = pl.pallas_call(
      _tc2b_body,
      grid=(_GRID,),
      in_specs=[_row_spec(4 * _H), _full_spec((2, 2 * _H)),
                _full_spec((1, 2 * _H)), _full_spec((1, 2 * _H)),
                _full_spec((2 * _H, _H)), _full_spec((2 * _H, _H))],
      out_specs=[_row_spec(2 * _H), _row_spec(2 * _H)],
      out_shape=[jax.ShapeDtypeStruct((_NP, 2 * _H), f32),
                 jax.ShapeDtypeStruct((_NP, 2 * _H), f32)],
  )(s0p, st0, gamma0_2, beta0_2, W_self1, W_neigh1)

  agg1 = segsum(t1p.reshape(4 * _ACC, _GW), src4, dst2)
  agg1p = agg1.reshape(_NP, 2 * _H)

  s1p, st1 = pl.pallas_call(
      _sage1_body,
      grid=(_GRID,),
      in_specs=[_row_spec(2 * _H), _row_spec(2 * _H), _row_spec(2 * _H),
                _full_spec((1, _H))],
      out_specs=[_row_spec(2 * _H), pl.BlockSpec((2, _H), lambda i: (0, 0))],
      out_shape=[jax.ShapeDtypeStruct((_NP, 2 * _H), f32),
                 jax.ShapeDtypeStruct((2, _H), f32)],
  )(self1p, agg1p, rpack, b_sage1_2)

  outp = pl.pallas_call(
      _tc3b_body,
      grid=(_GRID,),
      in_specs=[_row_spec(2 * _H), _full_spec((2, _H)), _full_spec((1, _H)),
                _full_spec((1, _H)), _row_spec(2 * _H),
                _full_spec((2 * _H, _H)), _full_spec((1, _H)),
                _full_spec((_H, _H // 2)), _full_spec((1, _H // 2)),
                _full_spec((_H // 2, 16)), _full_spec((1, 16))],
      out_specs=_row_spec(32),
      out_shape=jax.ShapeDtypeStruct((_NP, 32), f32),
  )(s1p, st1, gamma1_2, beta1_2, h0p, W_rel, b_rel[None, :], W_c1,
    b_c1[None, :], W_c2, b_c2[None, :])

  return outp.reshape(2 * _NP, 16)[:_N]


# speedup vs baseline: 9.6459x; 1.0024x over previous
import functools

import jax
import jax.numpy as jnp
from jax import lax
from jax.experimental import pallas as pl
from jax.experimental.pallas import tpu as pltpu
from jax.experimental.pallas import tpu_sc as plsc

_N = 50000
_H = 64
_NC = 2
_NS = 16
_CH = 128
_GW = 16
_ACC = 51200
_STRIPE = _ACC // _NS
_NP = _ACC // 2

_BP = 1600
_GRID = _NP // _BP
_EPS = 1e-5

_SC_PARAMS = pltpu.CompilerParams(use_tc_tiling_on_sc=False)

_NB = 7
_LEAD = 3


def _make_segsum(cpt, gpt):
  mesh = plsc.VectorSubcoreMesh(core_axis_name="c", subcore_axis_name="s")
  ngrp = cpt // gpt
  scratch = [
      pltpu.VMEM((gpt, _CH), jnp.int32),
      pltpu.VMEM((gpt, _CH), jnp.int32),
      pltpu.VMEM((_CH, _GW), jnp.float32),
      pltpu.VMEM_SHARED((_ACC, _GW), jnp.float32),
  ]
  scratch += [pltpu.VMEM((_CH, _GW), jnp.float32) for _ in range(_NB)]
  scratch += [pltpu.SemaphoreType.DMA for _ in range(2 * _NB)]

  def body(table, src2, dst2, agg_out, src_v, dst_v, zb, acc, *bufs):
    rows = bufs[:_NB]
    gsem = bufs[_NB:2 * _NB]
    ssem = bufs[2 * _NB:]
    c = lax.axis_index("c")
    s = lax.axis_index("s")
    base = s * _STRIPE

    @pl.loop(0, _CH)
    def _(r):
      zb[r, pl.ds(0, 16)] = jnp.zeros((16,), jnp.float32)

    def gather_wait(b):
      pltpu.make_async_copy(table.at[src_v.at[0]], rows[b], gsem[b]).wait()


    def scatter_wait(b):
      pltpu.make_async_copy(rows[b], acc.at[dst_v.at[0]], ssem[b]).wait()

    for p in range(2):
      tq = table.at[pl.ds(2 * p + c, 4 * _ACC - 3)]
      @pl.loop(0, _STRIPE // _CH)
      def _(r):
        pltpu.sync_copy(zb, acc.at[pl.ds(base + r * _CH, _CH)])
      plsc.subcore_barrier()

      @pl.loop(0, ngrp)
      def _(g):
        row0 = s * cpt + g * gpt
        pltpu.sync_copy(src2.at[pl.ds(row0, gpt)], src_v)
        pltpu.sync_copy(dst2.at[pl.ds(row0, gpt)], dst_v)

        for j in range(_LEAD):
          pltpu.async_copy(tq.at[src_v.at[j]], rows[j], gsem[j])

        @pl.loop(0, gpt // _NB)
        def _(jj):
          for t in range(_NB):
            j = jj * _NB + t
            gather_wait(t)
            pltpu.async_copy(rows[t], acc.at[dst_v.at[j]], ssem[t],
                             add=True)
            tn = (t + _LEAD) % _NB
            jn = j + _LEAD

            @pl.when(jn < gpt)
            def _():
              @pl.when(jn >= _NB)
              def _():
                scatter_wait(tn)
              pltpu.async_copy(tq.at[src_v.at[jn]], rows[tn], gsem[tn])

        for b in range(_NB):
          scatter_wait(b)

      plsc.subcore_barrier()
      pltpu.sync_copy(acc.at[pl.ds(base, _STRIPE)],
                      agg_out.at[pl.ds(base, _STRIPE), 2 * p + c])

  return pl.kernel(
      body, out_type=jax.ShapeDtypeStruct((_ACC, 4, _GW), jnp.float32),
      mesh=mesh, scratch_types=scratch, compiler_params=_SC_PARAMS)


def _make_deg(cpt, gpt):
  mesh = plsc.VectorSubcoreMesh(core_axis_name="c", subcore_axis_name="s")
  ngrp = cpt // gpt
  scratch = [
      pltpu.VMEM((gpt, _CH), jnp.int32),
      pltpu.VMEM((_CH,), jnp.float32),
      pltpu.VMEM((_CH,), jnp.float32),
      pltpu.VMEM_SHARED((_ACC,), jnp.float32),
  ]

  def body(dst2, deg_out, dst_v, ones, zrow, dacc):
    c = lax.axis_index("c")
    s = lax.axis_index("s")
    base = s * _STRIPE

    @pl.loop(0, _CH // 16)
    def _(r):
      ones[pl.ds(r * 16, 16)] = jnp.ones((16,), jnp.float32)
      zrow[pl.ds(r * 16, 16)] = jnp.zeros((16,), jnp.float32)

    @pl.loop(0, _STRIPE // _CH)
    def _(r):
      pltpu.sync_copy(zrow, dacc.at[pl.ds(base + r * _CH, _CH)])
    plsc.subcore_barrier()

    @pl.loop(0, ngrp)
    def _(g):
      pltpu.sync_copy(dst2.at[pl.ds(s * cpt + g * gpt, gpt)], dst_v)

      @pl.loop(0, gpt)
      def _(j):
        @pl.when((j % 2) == c)
        def _():
          pltpu.sync_copy(ones, dacc.at[dst_v.at[j]], add=True)

    plsc.subcore_barrier()
    pltpu.sync_copy(dacc.at[pl.ds(base, _STRIPE)],
                    deg_out.at[c].at[pl.ds(base, _STRIPE)])

  return pl.kernel(
      body, out_type=jax.ShapeDtypeStruct((_NC, _ACC), jnp.float32),
      mesh=mesh, scratch_types=scratch, compiler_params=_SC_PARAMS)


def _halves(x):
  return x[:, :_H], x[:, _H:]


def _dot(a, b):
  return jnp.dot(a, b, preferred_element_type=jnp.float32)


def _tc1_body(s_ref, m_ref, w_ref, b_ref, h0_ref):
  w = w_ref[...]
  ws, wm = w[:_H], w[_H:]
  b = b_ref[...]
  se, so = _halves(s_ref[...])
  me, mo = _halves(m_ref[...])
  he = jnp.maximum(_dot(se, ws) + _dot(me, wm) + b, 0.0)
  ho = jnp.maximum(_dot(so, ws) + _dot(mo, wm) + b, 0.0)
  h0_ref[...] = jnp.concatenate([he, ho], axis=1)


def _mask(i):
  rid = i * _BP + lax.broadcasted_iota(jnp.int32, (_BP, 1), 0)
  return rid < (_N // 2)


def _stats_update(i, st_ref, se, so):
  m = _mask(i)
  sem = jnp.where(m, se, 0.0)
  som = jnp.where(m, so, 0.0)

  @pl.when(i == 0)
  def _():
    st_ref[...] = jnp.zeros_like(st_ref)
  st_ref[...] += jnp.stack([
      jnp.sum(sem, axis=0) + jnp.sum(som, axis=0),
      jnp.sum(sem * sem, axis=0) + jnp.sum(som * som, axis=0)])


def _sage0_body(h0_ref, a_ref, rp_ref, ws_ref, wn_ref, b_ref,
                s_out_ref, st_ref):
  i = pl.program_id(0)
  hn = a_ref[...] * rp_ref[...]
  ae, ao = _halves(hn)
  h0e, h0o = _halves(h0_ref[...])
  ws, wn, b = ws_ref[...], wn_ref[...], b_ref[...]
  se = _dot(h0e, ws) + _dot(ae, wn) + b
  so = _dot(h0o, ws) + _dot(ao, wn) + b
  s_out_ref[...] = jnp.concatenate([se, so], axis=1)
  _stats_update(i, st_ref, se, so)


def _bnorm(x, st, g, be):
  mean = st[0:1] / _N
  var = st[1:2] / _N - mean * mean
  inv = lax.rsqrt(var + _EPS)
  return jnp.maximum((x - mean) * inv * g + be, 0.0)


def _tc2b_body(s_ref, st_ref, g_ref, be_ref, ws_ref, wn_ref,
               self1_ref, t1_ref):
  st, g, be = st_ref[...], g_ref[...], be_ref[...]
  ws, wn = ws_ref[...], wn_ref[...]
  s = s_ref[...]
  h1e = _bnorm(s[:, :2 * _H], st, g, be)
  h1o = _bnorm(s[:, 2 * _H:], st, g, be)
  self1_ref[...] = jnp.concatenate([_dot(h1e, ws), _dot(h1o, ws)], axis=1)
  t1_ref[...] = jnp.concatenate([_dot(h1e, wn), _dot(h1o, wn)], axis=1)


def _sage1_body(self1_ref, a_ref, rp_ref, b_ref, s_out_ref, st_ref):
  i = pl.program_id(0)
  b = b_ref[...]
  s = (self1_ref[...] + a_ref[...] * rp_ref[...]
       + jnp.concatenate([b, b], axis=1))
  s_out_ref[...] = s
  se, so = _halves(s)
  _stats_update(i, st_ref, se, so)


def _tc3b_body(s_ref, st_ref, g_ref, be_ref, h0_ref, wrel_ref, brel_ref,
               wc1_ref, bc1_ref, wc2_ref, bc2_ref, out_ref):
  st, g, be = st_ref[...], g_ref[...], be_ref[...]
  wrel, brel = wrel_ref[...], brel_ref[...]
  wra, wrb = wrel[:_H], wrel[_H:]
  wc1, bc1 = wc1_ref[...], bc1_ref[...]
  wc2, bc2 = wc2_ref[...], bc2_ref[...]
  s = s_ref[...]
  h0e, h0o = _halves(h0_ref[...])

  def head(h0h, sh):
    h2 = _bnorm(sh, st, g, be)
    hf = jnp.maximum(_dot(h0h, wra) + _dot(h2, wrb) + brel, 0.0)
    hid = jnp.maximum(_dot(hf, wc1) + bc1, 0.0)
    return _dot(hid, wc2) + bc2

  oe = head(h0e, s[:, :_H])
  oo = head(h0o, s[:, _H:])
  out_ref[...] = jnp.concatenate([oe, oo], axis=1)


def _row_spec(width):
  return pl.BlockSpec((_BP, width), lambda i: (i, 0))


def _full_spec(shape):
  nd = len(shape)
  return pl.BlockSpec(shape, lambda i, _n=nd: (0,) * _n)


def kernel(structural_features, multimodal_features, edge_index, W_in, b_in,
           W_self0, W_neigh0, b_sage0, gamma0, beta0, W_self1, W_neigh1,
           b_sage1, gamma1, beta1, W_rel, b_rel, W_c1, b_c1, W_c2, b_c2):
  f32 = jnp.float32
  src = edge_index[0]
  dst = edge_index[1]
  e = src.shape[0]

  gpt = 56
  cpt = -(-e // (_CH * _NS))
  cpt = -(-cpt // gpt) * gpt
  e_pad = cpt * _CH * _NS
  padn = e_pad - e
  srcp = jnp.concatenate([src, jnp.zeros((padn,), jnp.int32)])
  dstp = jnp.concatenate([dst, jnp.full((padn,), _N, jnp.int32)])
  src2 = (4 * srcp).reshape(e_pad // _CH, _CH)
  dst2 = dstp.reshape(e_pad // _CH, _CH)

  segsum = _make_segsum(cpt, gpt)
  degk = _make_deg(cpt, gpt)

  s128 = structural_features.reshape(_N // 2, 2 * _H)
  m128 = multimodal_features.reshape(_N // 2, 2 * _H)

  b_in2 = b_in[None, :]
  b_sage0_2 = b_sage0[None, :]
  b_sage1_2 = b_sage1[None, :]
  gamma0_2, beta0_2 = gamma0[None, :], beta0[None, :]
  gamma1_2, beta1_2 = gamma1[None, :], beta1[None, :]

  degp = degk(dst2)
  rpack = jnp.broadcast_to(
      (1.0 / jnp.maximum(degp[0] + degp[1], 1.0))[:, None],
      (_ACC, _H)).reshape(_NP, 2 * _H)

  h0p = pl.pallas_call(
      _tc1_body,
      grid=(_GRID,),
      in_specs=[_row_spec(2 * _H), _row_spec(2 * _H),
                _full_spec((2 * _H, _H)), _full_spec((1, _H))],
      out_specs=_row_spec(2 * _H),
      out_shape=jax.ShapeDtypeStruct((_NP, 2 * _H), f32),
  )(s128, m128, W_in, b_in2)

  agg0 = segsum(h0p.reshape(4 * _ACC, _GW), src2, dst2)
  agg0p = agg0.reshape(_NP, 2 * _H)

  s0p, st0 = pl.pallas_call(
      _sage0_body,
      grid=(_GRID,),
      in_specs=[_row_spec(2 * _H), _row_spec(2 * _H), _row_spec(2 * _H),
                _full_spec((_H, 2 * _H)), _full_spec((_H, 2 * _H)),
                _full_spec((1, 2 * _H))],
      out_specs=[_row_spec(4 * _H),
                 pl.BlockSpec((2, 2 * _H), lambda i: (0, 0))],
      out_shape=[jax.ShapeDtypeStruct((_NP, 4 * _H), f32),
                 jax.ShapeDtypeStruct((2, 2 * _H), f32)],
  )(h0p, agg0p, rpack, W_self0, W_neigh0, b_sage0_2)

  self1p, t1p = pl.pallas_call(
      _tc2b_body,
      grid=(_GRID,),
      in_specs=[_row_spec(4 * _H), _full_spec((2, 2 * _H)),
                _full_spec((1, 2 * _H)), _full_spec((1, 2 * _H)),
                _full_spec((2 * _H, _H)), _full_spec((2 * _H, _H))],
      out_specs=[_row_spec(2 * _H), _row_spec(2 * _H)],
      out_shape=[jax.ShapeDtypeStruct((_NP, 2 * _H), f32),
                 jax.ShapeDtypeStruct((_NP, 2 * _H), f32)],
  )(s0p, st0, gamma0_2, beta0_2, W_self1, W_neigh1)

  agg1 = segsum(t1p.reshape(4 * _ACC, _GW), src2, dst2)
  agg1p = agg1.reshape(_NP, 2 * _H)

  s1p, st1 = pl.pallas_call(
      _sage1_body,
      grid=(_GRID,),
      in_specs=[_row_spec(2 * _H), _row_spec(2 * _H), _row_spec(2 * _H),
                _full_spec((1, _H))],
      out_specs=[_row_spec(2 * _H), pl.BlockSpec((2, _H), lambda i: (0, 0))],
      out_shape=[jax.ShapeDtypeStruct((_NP, 2 * _H), f32),
                 jax.ShapeDtypeStruct((2, _H), f32)],
  )(self1p, agg1p, rpack, b_sage1_2)

  outp = pl.pallas_call(
      _tc3b_body,
      grid=(_GRID,),
      in_specs=[_row_spec(2 * _H), _full_spec((2, _H)), _full_spec((1, _H)),
                _full_spec((1, _H)), _row_spec(2 * _H),
                _full_spec((2 * _H, _H)), _full_spec((1, _H)),
                _full_spec((_H, _H // 2)), _full_spec((1, _H // 2)),
                _full_spec((_H // 2, 16)), _full_spec((1, 16))],
      out_specs=_row_spec(32),
      out_shape=jax.ShapeDtypeStruct((_NP, 32), f32),
  )(s1p, st1, gamma1_2, beta1_2, h0p, W_rel, b_rel[None, :], W_c1,
    b_c1[None, :], W_c2, b_c2[None, :])

  return outp.reshape(2 * _NP, 16)[:_N]


# speedup vs baseline: 11.7907x; 1.2224x over previous
import functools

import jax
import jax.numpy as jnp
from jax import lax
from jax.experimental import pallas as pl
from jax.experimental.pallas import tpu as pltpu
from jax.experimental.pallas import tpu_sc as plsc

_N = 50000
_H = 64
_NC = 2
_NS = 16
_CH = 128
_GW = 16
_ACC = 51200
_STRIPE = _ACC // _NS
_NP = _ACC // 2

_BP = 1600
_GRID = _NP // _BP
_EPS = 1e-5

_SC_PARAMS = pltpu.CompilerParams(use_tc_tiling_on_sc=False)

_NB = 14
_LEAD = 7


def _make_segsum(cpt, gpt):
  mesh = plsc.VectorSubcoreMesh(core_axis_name="c", subcore_axis_name="s")
  ngrp = cpt // gpt
  scratch = [
      pltpu.VMEM((gpt, _CH), jnp.int32),
      pltpu.VMEM((gpt, _CH), jnp.int32),
      pltpu.VMEM((_CH, _GW), jnp.float32),
      pltpu.VMEM_SHARED((_ACC, _GW), jnp.float32),
  ]
  scratch += [pltpu.VMEM((_CH, _GW), jnp.float32) for _ in range(_NB)]
  scratch += [pltpu.SemaphoreType.DMA for _ in range(2 * _NB)]

  def body(table, src2, dst2, agg_out, src_v, dst_v, zb, acc, *bufs):
    rows = bufs[:_NB]
    gsem = bufs[_NB:2 * _NB]
    ssem = bufs[2 * _NB:]
    c = lax.axis_index("c")
    s = lax.axis_index("s")
    base = s * _STRIPE

    @pl.loop(0, _CH)
    def _(r):
      zb[r, pl.ds(0, 16)] = jnp.zeros((16,), jnp.float32)

    def gather_wait(b):
      pltpu.make_async_copy(table.at[src_v.at[0]], rows[b], gsem[b]).wait()


    def scatter_wait(b):
      pltpu.make_async_copy(rows[b], acc.at[dst_v.at[0]], ssem[b]).wait()

    for p in range(2):
      tq = table.at[pl.ds(2 * p + c, 4 * _ACC - 3)]
      @pl.loop(0, _STRIPE // _CH)
      def _(r):
        pltpu.sync_copy(zb, acc.at[pl.ds(base + r * _CH, _CH)])
      plsc.subcore_barrier()

      @pl.loop(0, ngrp)
      def _(g):
        row0 = s * cpt + g * gpt
        pltpu.sync_copy(src2.at[pl.ds(row0, gpt)], src_v)
        pltpu.sync_copy(dst2.at[pl.ds(row0, gpt)], dst_v)

        for j in range(_LEAD):
          pltpu.async_copy(tq.at[src_v.at[j]], rows[j], gsem[j])

        @pl.loop(0, gpt // _NB)
        def _(jj):
          for t in range(_NB):
            j = jj * _NB + t
            gather_wait(t)
            pltpu.async_copy(rows[t], acc.at[dst_v.at[j]], ssem[t],
                             add=True)
            tn = (t + _LEAD) % _NB
            jn = j + _LEAD

            @pl.when(jn < gpt)
            def _():
              @pl.when(jn >= _NB)
              def _():
                scatter_wait(tn)
              pltpu.async_copy(tq.at[src_v.at[jn]], rows[tn], gsem[tn])

        for b in range(_NB):
          scatter_wait(b)

      plsc.subcore_barrier()
      pltpu.sync_copy(acc.at[pl.ds(base, _STRIPE)],
                      agg_out.at[pl.ds(base, _STRIPE), 2 * p + c])

  return pl.kernel(
      body, out_type=jax.ShapeDtypeStruct((_ACC, 4, _GW), jnp.float32),
      mesh=mesh, scratch_types=scratch, compiler_params=_SC_PARAMS)


def _make_deg(cpt, gpt):
  mesh = plsc.VectorSubcoreMesh(core_axis_name="c", subcore_axis_name="s")
  ngrp = cpt // gpt
  scratch = [
      pltpu.VMEM((gpt, _CH), jnp.int32),
      pltpu.VMEM((_CH,), jnp.float32),
      pltpu.VMEM((_CH,), jnp.float32),
      pltpu.VMEM_SHARED((_ACC,), jnp.float32),
  ]

  def body(dst2, deg_out, dst_v, ones, zrow, dacc):
    c = lax.axis_index("c")
    s = lax.axis_index("s")
    base = s * _STRIPE

    @pl.loop(0, _CH // 16)
    def _(r):
      ones[pl.ds(r * 16, 16)] = jnp.ones((16,), jnp.float32)
      zrow[pl.ds(r * 16, 16)] = jnp.zeros((16,), jnp.float32)

    @pl.loop(0, _STRIPE // _CH)
    def _(r):
      pltpu.sync_copy(zrow, dacc.at[pl.ds(base + r * _CH, _CH)])
    plsc.subcore_barrier()

    @pl.loop(0, ngrp)
    def _(g):
      pltpu.sync_copy(dst2.at[pl.ds(s * cpt + g * gpt, gpt)], dst_v)

      @pl.loop(0, gpt)
      def _(j):
        @pl.when((j % 2) == c)
        def _():
          pltpu.sync_copy(ones, dacc.at[dst_v.at[j]], add=True)

    plsc.subcore_barrier()
    pltpu.sync_copy(dacc.at[pl.ds(base, _STRIPE)],
                    deg_out.at[c].at[pl.ds(base, _STRIPE)])

  return pl.kernel(
      body, out_type=jax.ShapeDtypeStruct((_NC, _ACC), jnp.float32),
      mesh=mesh, scratch_types=scratch, compiler_params=_SC_PARAMS)


def _halves(x):
  return x[:, :_H], x[:, _H:]


def _dot(a, b):
  return jnp.dot(a, b, preferred_element_type=jnp.float32)


def _tc1_body(s_ref, m_ref, w_ref, b_ref, h0_ref):
  w = w_ref[...]
  ws, wm = w[:_H], w[_H:]
  b = b_ref[...]
  se, so = _halves(s_ref[...])
  me, mo = _halves(m_ref[...])
  he = jnp.maximum(_dot(se, ws) + _dot(me, wm) + b, 0.0)
  ho = jnp.maximum(_dot(so, ws) + _dot(mo, wm) + b, 0.0)
  h0_ref[...] = jnp.concatenate([he, ho], axis=1)


def _mask(i):
  rid = i * _BP + lax.broadcasted_iota(jnp.int32, (_BP, 1), 0)
  return rid < (_N // 2)


def _stats_update(i, st_ref, se, so):
  m = _mask(i)
  sem = jnp.where(m, se, 0.0)
  som = jnp.where(m, so, 0.0)

  @pl.when(i == 0)
  def _():
    st_ref[...] = jnp.zeros_like(st_ref)
  st_ref[...] += jnp.stack([
      jnp.sum(sem, axis=0) + jnp.sum(som, axis=0),
      jnp.sum(sem * sem, axis=0) + jnp.sum(som * som, axis=0)])


def _sage0_body(h0_ref, a_ref, rp_ref, ws_ref, wn_ref, b_ref,
                s_out_ref, st_ref):
  i = pl.program_id(0)
  hn = a_ref[...] * rp_ref[...]
  ae, ao = _halves(hn)
  h0e, h0o = _halves(h0_ref[...])
  ws, wn, b = ws_ref[...], wn_ref[...], b_ref[...]
  se = _dot(h0e, ws) + _dot(ae, wn) + b
  so = _dot(h0o, ws) + _dot(ao, wn) + b
  s_out_ref[...] = jnp.concatenate([se, so], axis=1)
  _stats_update(i, st_ref, se, so)


def _bnorm(x, st, g, be):
  mean = st[0:1] / _N
  var = st[1:2] / _N - mean * mean
  inv = lax.rsqrt(var + _EPS)
  return jnp.maximum((x - mean) * inv * g + be, 0.0)


def _tc2b_body(s_ref, st_ref, g_ref, be_ref, ws_ref, wn_ref,
               self1_ref, t1_ref):
  st, g, be = st_ref[...], g_ref[...], be_ref[...]
  ws, wn = ws_ref[...], wn_ref[...]
  s = s_ref[...]
  h1e = _bnorm(s[:, :2 * _H], st, g, be)
  h1o = _bnorm(s[:, 2 * _H:], st, g, be)
  self1_ref[...] = jnp.concatenate([_dot(h1e, ws), _dot(h1o, ws)], axis=1)
  t1_ref[...] = jnp.concatenate([_dot(h1e, wn), _dot(h1o, wn)], axis=1)


def _sage1_body(self1_ref, a_ref, rp_ref, b_ref, s_out_ref, st_ref):
  i = pl.program_id(0)
  b = b_ref[...]
  s = (self1_ref[...] + a_ref[...] * rp_ref[...]
       + jnp.concatenate([b, b], axis=1))
  s_out_ref[...] = s
  se, so = _halves(s)
  _stats_update(i, st_ref, se, so)


def _tc3b_body(s_ref, st_ref, g_ref, be_ref, h0_ref, wrel_ref, brel_ref,
               wc1_ref, bc1_ref, wc2_ref, bc2_ref, out_ref):
  st, g, be = st_ref[...], g_ref[...], be_ref[...]
  wrel, brel = wrel_ref[...], brel_ref[...]
  wra, wrb = wrel[:_H], wrel[_H:]
  wc1, bc1 = wc1_ref[...], bc1_ref[...]
  wc2, bc2 = wc2_ref[...], bc2_ref[...]
  s = s_ref[...]
  h0e, h0o = _halves(h0_ref[...])

  def head(h0h, sh):
    h2 = _bnorm(sh, st, g, be)
    hf = jnp.maximum(_dot(h0h, wra) + _dot(h2, wrb) + brel, 0.0)
    hid = jnp.maximum(_dot(hf, wc1) + bc1, 0.0)
    return _dot(hid, wc2) + bc2

  oe = head(h0e, s[:, :_H])
  oo = head(h0o, s[:, _H:])
  out_ref[...] = jnp.concatenate([oe, oo], axis=1)


def _row_spec(width):
  return pl.BlockSpec((_BP, width), lambda i: (i, 0))


def _full_spec(shape):
  nd = len(shape)
  return pl.BlockSpec(shape, lambda i, _n=nd: (0,) * _n)


def kernel(structural_features, multimodal_features, edge_index, W_in, b_in,
           W_self0, W_neigh0, b_sage0, gamma0, beta0, W_self1, W_neigh1,
           b_sage1, gamma1, beta1, W_rel, b_rel, W_c1, b_c1, W_c2, b_c2):
  f32 = jnp.float32
  src = edge_index[0]
  dst = edge_index[1]
  e = src.shape[0]

  gpt = 56
  cpt = -(-e // (_CH * _NS))
  cpt = -(-cpt // gpt) * gpt
  e_pad = cpt * _CH * _NS
  padn = e_pad - e
  srcp = jnp.concatenate([src, jnp.zeros((padn,), jnp.int32)])
  dstp = jnp.concatenate([dst, jnp.full((padn,), _N, jnp.int32)])
  src2 = (4 * srcp).reshape(e_pad // _CH, _CH)
  dst2 = dstp.reshape(e_pad // _CH, _CH)

  segsum = _make_segsum(cpt, gpt)
  degk = _make_deg(cpt, gpt)

  s128 = structural_features.reshape(_N // 2, 2 * _H)
  m128 = multimodal_features.reshape(_N // 2, 2 * _H)

  b_in2 = b_in[None, :]
  b_sage0_2 = b_sage0[None, :]
  b_sage1_2 = b_sage1[None, :]
  gamma0_2, beta0_2 = gamma0[None, :], beta0[None, :]
  gamma1_2, beta1_2 = gamma1[None, :], beta1[None, :]

  degp = degk(dst2)
  rpack = jnp.broadcast_to(
      (1.0 / jnp.maximum(degp[0] + degp[1], 1.0))[:, None],
      (_ACC, _H)).reshape(_NP, 2 * _H)

  h0p = pl.pallas_call(
      _tc1_body,
      grid=(_GRID,),
      in_specs=[_row_spec(2 * _H), _row_spec(2 * _H),
                _full_spec((2 * _H, _H)), _full_spec((1, _H))],
      out_specs=_row_spec(2 * _H),
      out_shape=jax.ShapeDtypeStruct((_NP, 2 * _H), f32),
  )(s128, m128, W_in, b_in2)

  agg0 = segsum(h0p.reshape(4 * _ACC, _GW), src2, dst2)
  agg0p = agg0.reshape(_NP, 2 * _H)

  s0p, st0 = pl.pallas_call(
      _sage0_body,
      grid=(_GRID,),
      in_specs=[_row_spec(2 * _H), _row_spec(2 * _H), _row_spec(2 * _H),
                _full_spec((_H, 2 * _H)), _full_spec((_H, 2 * _H)),
                _full_spec((1, 2 * _H))],
      out_specs=[_row_spec(4 * _H),
                 pl.BlockSpec((2, 2 * _H), lambda i: (0, 0))],
      out_shape=[jax.ShapeDtypeStruct((_NP, 4 * _H), f32),
                 jax.ShapeDtypeStruct((2, 2 * _H), f32)],
  )(h0p, agg0p, rpack, W_self0, W_neigh0, b_sage0_2)

  self1p, t1p = pl.pallas_call(
      _tc2b_body,
      grid=(_GRID,),
      in_specs=[_row_spec(4 * _H), _full_spec((2, 2 * _H)),
                _full_spec((1, 2 * _H)), _full_spec((1, 2 * _H)),
                _full_spec((2 * _H, _H)), _full_spec((2 * _H, _H))],
      out_specs=[_row_spec(2 * _H), _row_spec(2 * _H)],
      out_shape=[jax.ShapeDtypeStruct((_NP, 2 * _H), f32),
                 jax.ShapeDtypeStruct((_NP, 2 * _H), f32)],
  )(s0p, st0, gamma0_2, beta0_2, W_self1, W_neigh1)

  agg1 = segsum(t1p.reshape(4 * _ACC, _GW), src2, dst2)
  agg1p = agg1.reshape(_NP, 2 * _H)

  s1p, st1 = pl.pallas_call(
      _sage1_body,
      grid=(_GRID,),
      in_specs=[_row_spec(2 * _H), _row_spec(2 * _H), _row_spec(2 * _H),
                _full_spec((1, _H))],
      out_specs=[_row_spec(2 * _H), pl.BlockSpec((2, _H), lambda i: (0, 0))],
      out_shape=[jax.ShapeDtypeStruct((_NP, 2 * _H), f32),
                 jax.ShapeDtypeStruct((2, _H), f32)],
  )(self1p, agg1p, rpack, b_sage1_2)

  outp = pl.pallas_call(
      _tc3b_body,
      grid=(_GRID,),
      in_specs=[_row_spec(2 * _H), _full_spec((2, _H)), _full_spec((1, _H)),
                _full_spec((1, _H)), _row_spec(2 * _H),
                _full_spec((2 * _H, _H)), _full_spec((1, _H)),
                _full_spec((_H, _H // 2)), _full_spec((1, _H // 2)),
                _full_spec((_H // 2, 16)), _full_spec((1, 16))],
      out_specs=_row_spec(32),
      out_shape=jax.ShapeDtypeStruct((_NP, 32), f32),
  )(s1p, st1, gamma1_2, beta1_2, h0p, W_rel, b_rel[None, :], W_c1,
    b_c1[None, :], W_c2, b_c2[None, :])

  return outp.reshape(2 * _NP, 16)[:_N]
